# Initial kernel scaffold; baseline (speedup 1.0000x reference)
import jax
import jax.numpy as jnp
from jax.experimental import pallas as pl


def kernel(inputs, table):
    raise NotImplementedError("write your pallas kernel here")


# speedup vs baseline: 1.4436x; 1.4436x over previous
import jax
import jax.numpy as jnp
from jax.experimental import pallas as pl

_BS = 512


def _add_body(x_ref, t_ref, o_ref):
    o_ref[...] = x_ref[...] + t_ref[...]


def kernel(inputs, table):
    B, S, D = inputs.shape
    grid = (S // _BS, B)
    return pl.pallas_call(
        _add_body,
        grid=grid,
        in_specs=[
            pl.BlockSpec((1, _BS, D), lambda i, j: (j, i, 0)),
            pl.BlockSpec((_BS, D), lambda i, j: (i, 0)),
        ],
        out_specs=pl.BlockSpec((1, _BS, D), lambda i, j: (j, i, 0)),
        out_shape=jax.ShapeDtypeStruct((B, S, D), inputs.dtype),
    )(inputs, table)


# speedup vs baseline: 1.6808x; 1.1643x over previous
import jax
import jax.numpy as jnp
from jax.experimental import pallas as pl

_BS = 1024


def _add_body(x_ref, t_ref, o_ref):
    o_ref[...] = x_ref[...] + t_ref[...]


def kernel(inputs, table):
    B, S, D = inputs.shape
    grid = (S // _BS, B)
    return pl.pallas_call(
        _add_body,
        grid=grid,
        in_specs=[
            pl.BlockSpec((1, _BS, D), lambda i, j: (j, i, 0)),
            pl.BlockSpec((_BS, D), lambda i, j: (i, 0)),
        ],
        out_specs=pl.BlockSpec((1, _BS, D), lambda i, j: (j, i, 0)),
        out_shape=jax.ShapeDtypeStruct((B, S, D), inputs.dtype),
    )(inputs, table)


# speedup vs baseline: 1.7988x; 1.0702x over previous
import jax
import jax.numpy as jnp
from jax.experimental import pallas as pl

_BS = 2048


def _add_body(x_ref, t_ref, o_ref):
    o_ref[...] = x_ref[...] + t_ref[...]


def kernel(inputs, table):
    B, S, D = inputs.shape
    grid = (S // _BS, B)
    return pl.pallas_call(
        _add_body,
        grid=grid,
        in_specs=[
            pl.BlockSpec((1, _BS, D), lambda i, j: (j, i, 0)),
            pl.BlockSpec((_BS, D), lambda i, j: (i, 0)),
        ],
        out_specs=pl.BlockSpec((1, _BS, D), lambda i, j: (j, i, 0)),
        out_shape=jax.ShapeDtypeStruct((B, S, D), inputs.dtype),
    )(inputs, table)
